# baseline (speedup 1.0000x reference)
import jax
import jax.numpy as jnp
import numpy as np
from jax import lax
from jax.experimental import pallas as pl
from jax.experimental.pallas import tpu as pltpu
from jax.experimental.pallas import tpu_sc as plsc

B, AT, NBR, NFEAT = 2, 512, 16, 128
NM = NBR - 1
RPA = NBR * NM
NC, NS = 2, 16
NW = NC * NS
NATOMS = B * AT
APW = NATOMS // NW
NV = NFEAT // 16

_DESTS = [
    [i * NM + (k if k < i else k - 1) for i in range(NBR) if i != k]
    for k in range(NBR)
]


def _sc_body(emb_hbm, nbr_hbm, out_hbm, nbr_v, rows_v, obuf_v, gsem, wsem):
    wid = lax.axis_index("s") * NC + lax.axis_index("c")
    base = wid * APW
    pltpu.sync_copy(nbr_hbm.at[pl.ds(base, APW)], nbr_v)
    for a in range(APW):
        pltpu.async_copy(emb_hbm.at[nbr_v.at[a]], rows_v.at[a], gsem)
    for a in range(APW):
        pltpu.make_async_copy(emb_hbm.at[nbr_v.at[a]], rows_v.at[a], gsem).wait()

    def expand_and_send(a, slot, first):
        @pl.when(jnp.logical_not(first))
        def _():
            pltpu.make_async_copy(obuf_v.at[slot], out_hbm.at[0], wsem).wait()

        for k in range(NBR):
            regs = [rows_v[a, k, pl.ds(c * 16, 16)] for c in range(NV)]
            for j in _DESTS[k]:
                for c in range(NV):
                    obuf_v[slot, j, pl.ds(c * 16, 16)] = regs[c]
        pltpu.async_copy(obuf_v.at[slot], out_hbm.at[base + a], wsem)

    def pair_body(t, carry):
        a0 = t * 2
        expand_and_send(a0, 0, t == 0)
        expand_and_send(a0 + 1, 1, t == 0)
        return carry

    lax.fori_loop(0, APW // 2, pair_body, 0)
    pltpu.make_async_copy(obuf_v.at[0], out_hbm.at[0], wsem).wait()
    pltpu.make_async_copy(obuf_v.at[1], out_hbm.at[0], wsem).wait()


def kernel(node_embedding, nbr_idx):
    emb_flat = node_embedding.reshape(NATOMS, NFEAT)
    batch_off = (jnp.arange(B, dtype=jnp.int32) * AT)[:, None, None]
    nbr_glob = (nbr_idx.astype(jnp.int32) + batch_off).reshape(NATOMS, NBR)

    run = pl.kernel(
        _sc_body,
        out_type=jax.ShapeDtypeStruct((NATOMS, RPA, NFEAT), jnp.float32),
        mesh=plsc.VectorSubcoreMesh(core_axis_name="c", subcore_axis_name="s"),
        scratch_types=[
            pltpu.VMEM((APW, NBR), jnp.int32),
            pltpu.VMEM((APW, NBR, NFEAT), jnp.float32),
            pltpu.VMEM((2, RPA, NFEAT), jnp.float32),
            pltpu.SemaphoreType.DMA,
            pltpu.SemaphoreType.DMA,
        ],
        compiler_params=pltpu.CompilerParams(
            needs_layout_passes=False, use_tc_tiling_on_sc=False
        ),
    )
    out = run(emb_flat, nbr_glob)
    return out.reshape(B, AT, NBR, NM, NFEAT)


# speedup vs baseline: 1.2912x; 1.2912x over previous
import jax
import jax.numpy as jnp
from jax import lax
from jax.experimental import pallas as pl
from jax.experimental.pallas import tpu as pltpu
from jax.experimental.pallas import tpu_sc as plsc

B, AT, NBR, NFEAT = 2, 512, 16, 128
NM = NBR - 1
RPA = NBR * NM
NDUP = 2 * NBR - 1
NC, NS = 2, 16
NW = NC * NS
NATOMS = B * AT
APW = NATOMS // NW


def _sc_body(emb_hbm, nbr_hbm, out_hbm, nbr_v, rows_v, gsem, wsem):
    wid = lax.axis_index("s") * NC + lax.axis_index("c")
    base = wid * APW
    pltpu.sync_copy(nbr_hbm.at[pl.ds(base, APW)], nbr_v)
    for a in range(APW):
        pltpu.async_copy(emb_hbm.at[nbr_v.at[a]], rows_v.at[a], gsem)
    for a in range(APW):
        pltpu.make_async_copy(emb_hbm.at[nbr_v.at[a]], rows_v.at[a], gsem).wait()
    for g in range(NM):
        pltpu.async_copy(
            rows_v.at[:, pl.ds(g + 1, NBR)],
            out_hbm.at[pl.ds(base, APW), pl.ds(g * NBR, NBR)],
            wsem,
        )
    for g in range(NM):
        pltpu.make_async_copy(
            rows_v.at[:, pl.ds(g + 1, NBR)],
            out_hbm.at[pl.ds(base, APW), pl.ds(g * NBR, NBR)],
            wsem,
        ).wait()


def kernel(node_embedding, nbr_idx):
    emb_flat = node_embedding.reshape(NATOMS, NFEAT)
    batch_off = (jnp.arange(B, dtype=jnp.int32) * AT)[:, None, None]
    nbr_glob = nbr_idx.astype(jnp.int32) + batch_off
    nbr_dup = jnp.concatenate([nbr_glob, nbr_glob[:, :, : NBR - 1]], axis=-1)
    nbr_dup = nbr_dup.reshape(NATOMS, NDUP)

    run = pl.kernel(
        _sc_body,
        out_type=jax.ShapeDtypeStruct((NATOMS, RPA, NFEAT), jnp.float32),
        mesh=plsc.VectorSubcoreMesh(core_axis_name="c", subcore_axis_name="s"),
        scratch_types=[
            pltpu.VMEM((APW, NDUP), jnp.int32),
            pltpu.VMEM((APW, NDUP, NFEAT), jnp.float32),
            pltpu.SemaphoreType.DMA,
            pltpu.SemaphoreType.DMA,
        ],
        compiler_params=pltpu.CompilerParams(
            needs_layout_passes=False, use_tc_tiling_on_sc=False
        ),
    )
    out = run(emb_flat, nbr_dup)
    return out.reshape(B, AT, NBR, NM, NFEAT)
